# baseline (speedup 1.0000x reference)
import functools

import jax
import jax.numpy as jnp
from jax.experimental import pallas as pl
from jax.experimental.pallas import tpu as pltpu

T = 2048
D = 1024
E = 8
K = 2
I = 512
ISH = 1024
G = 2
RSF = 2.5

NEG = -1e30


def _router_body(x_ref, gwt_ref, bias_ref, comb_ref, idx_ref):
    logits = jnp.dot(x_ref[...], gwt_ref[...],
                     preferred_element_type=jnp.float32)[:, :E]
    scores = jax.nn.sigmoid(logits)
    sc = scores + bias_ref[...]
    B = scores.shape[0]

    def top2sum(g):
        s = None
        for i in range(4):
            for j in range(i + 1, 4):
                p = g[:, i:i + 1] + g[:, j:j + 1]
                s = p if s is None else jnp.maximum(s, p)
        return s

    gs0 = top2sum(sc[:, 0:4])
    gs1 = top2sum(sc[:, 4:8])
    chosen = jnp.where(gs0 >= gs1, 0, 1)
    lane = jax.lax.broadcasted_iota(jnp.int32, (B, E), 1)
    emask = (lane // 4) == chosen
    masked = jnp.where(emask, sc, NEG)
    m1 = jnp.max(masked, axis=1, keepdims=True)
    i1 = jnp.min(jnp.where(masked == m1, lane, E), axis=1, keepdims=True)
    masked2 = jnp.where(lane == i1, NEG, masked)
    m2 = jnp.max(masked2, axis=1, keepdims=True)
    i2 = jnp.min(jnp.where(masked2 == m2, lane, E), axis=1, keepdims=True)
    selmask = jnp.logical_or(lane == i1, lane == i2)
    wsel = jnp.where(selmask, scores, 0.0)
    wsum = jnp.sum(wsel, axis=1, keepdims=True) + 1e-20
    comb_ref[...] = wsel * (RSF / wsum)
    idx_ref[...] = jnp.concatenate([i1, i2], axis=1)


def _router(x, gate_w, bias):
    gwt = jnp.zeros((D, 128), jnp.float32).at[:, :E].set(gate_w.T)
    bias2 = bias.reshape(1, E)
    BT = 512
    return pl.pallas_call(
        _router_body,
        grid=(T // BT,),
        in_specs=[
            pl.BlockSpec((BT, D), lambda b: (b, 0)),
            pl.BlockSpec((D, 128), lambda b: (0, 0)),
            pl.BlockSpec((1, E), lambda b: (0, 0)),
        ],
        out_specs=[
            pl.BlockSpec((BT, E), lambda b: (b, 0)),
            pl.BlockSpec((BT, K), lambda b: (b, 0)),
        ],
        out_shape=[
            jax.ShapeDtypeStruct((T, E), jnp.float32),
            jax.ShapeDtypeStruct((T, K), jnp.int32),
        ],
    )(x, gwt, bias2)


def _shared_body(x_ref, wgu_ref, wd_ref, out_ref):
    gu = jnp.dot(x_ref[...].astype(jnp.bfloat16), wgu_ref[...].astype(jnp.bfloat16),
                 preferred_element_type=jnp.float32)
    g = gu[:, :ISH]
    u = gu[:, ISH:]
    h = jax.nn.silu(g) * u
    out_ref[...] = jnp.dot(h.astype(jnp.bfloat16), wd_ref[...].astype(jnp.bfloat16),
                           preferred_element_type=jnp.float32)


def _shared(x, w_gu_t, w_d_t):
    BT = 512
    return pl.pallas_call(
        _shared_body,
        grid=(T // BT,),
        in_specs=[
            pl.BlockSpec((BT, D), lambda b: (b, 0)),
            pl.BlockSpec((D, 2 * ISH), lambda b: (0, 0)),
            pl.BlockSpec((ISH, D), lambda b: (0, 0)),
        ],
        out_specs=pl.BlockSpec((BT, D), lambda b: (b, 0)),
        out_shape=jax.ShapeDtypeStruct((T, D), jnp.float32),
    )(x, w_gu_t, w_d_t)


def _routed_body(x_ref, wgu_ref, wd_ref, comb_ref, shared_ref, out_ref, acc_ref):
    e = pl.program_id(1)
    lane = jax.lax.broadcasted_iota(jnp.int32, comb_ref.shape, 1)
    col = jnp.sum(jnp.where(lane == e, comb_ref[...], 0.0), axis=1,
                  keepdims=True)
    gu = jnp.dot(x_ref[...].astype(jnp.bfloat16), wgu_ref[0].astype(jnp.bfloat16),
                 preferred_element_type=jnp.float32)
    g = gu[:, :I]
    u = gu[:, I:]
    h = jax.nn.silu(g) * u * col
    y = jnp.dot(h.astype(jnp.bfloat16), wd_ref[0].astype(jnp.bfloat16),
                preferred_element_type=jnp.float32)

    @pl.when(e == 0)
    def _():
        acc_ref[...] = shared_ref[...] + y

    @pl.when(e > 0)
    def _():
        acc_ref[...] = acc_ref[...] + y

    @pl.when(e == E - 1)
    def _():
        out_ref[...] = acc_ref[...]


def _routed(x, w_gu_t, w_d_t, comb, shared_out):
    BT = 256
    return pl.pallas_call(
        _routed_body,
        grid=(T // BT, E),
        in_specs=[
            pl.BlockSpec((BT, D), lambda b, e: (b, 0)),
            pl.BlockSpec((1, D, 2 * I), lambda b, e: (e, 0, 0)),
            pl.BlockSpec((1, I, D), lambda b, e: (e, 0, 0)),
            pl.BlockSpec((BT, E), lambda b, e: (b, 0)),
            pl.BlockSpec((BT, D), lambda b, e: (b, 0)),
        ],
        out_specs=pl.BlockSpec((BT, D), lambda b, e: (b, 0)),
        out_shape=jax.ShapeDtypeStruct((T, D), jnp.float32),
        scratch_shapes=[pltpu.VMEM((BT, D), jnp.float32)],
    )(x, w_gu_t, w_d_t, comb, shared_out)


def kernel(x, max_num_tokens_per_gpu, gate_w, e_score_correction_bias,
           w_shared_gate_up, w_shared_down, w_expert_gate_up, w_expert_down):
    comb, _ = _router(x, gate_w, e_score_correction_bias)
    shared_out = _shared(x, w_shared_gate_up.T, w_shared_down.T)
    wgu_t = jnp.transpose(w_expert_gate_up, (0, 2, 1))
    wd_t = jnp.transpose(w_expert_down, (0, 2, 1))
    return _routed(x, wgu_t, wd_t, comb, shared_out)


# speedup vs baseline: 1.8476x; 1.8476x over previous
import jax
import jax.numpy as jnp
from jax.experimental import pallas as pl
from jax.experimental.pallas import tpu as pltpu

T = 2048
D = 1024
E = 8
K = 2
I = 512
ISH = 1024
RSF = 2.5

NEG = -1e30
BF = jnp.bfloat16
F32 = jnp.float32

NT = (((1,), (1,)), ((), ()))


def _router_body(x_ref, gwt_ref, bias_ref, comb_ref, idx_ref):
    logits = jnp.dot(x_ref[...], gwt_ref[...],
                     preferred_element_type=F32)[:, :E]
    scores = jax.nn.sigmoid(logits)
    sc = scores + bias_ref[...]
    B = scores.shape[0]

    def top2sum(g):
        s = None
        for i in range(4):
            for j in range(i + 1, 4):
                p = g[:, i:i + 1] + g[:, j:j + 1]
                s = p if s is None else jnp.maximum(s, p)
        return s

    gs0 = top2sum(sc[:, 0:4])
    gs1 = top2sum(sc[:, 4:8])
    chosen = jnp.where(gs0 >= gs1, 0, 1)
    lane = jax.lax.broadcasted_iota(jnp.int32, (B, E), 1)
    emask = (lane // 4) == chosen
    masked = jnp.where(emask, sc, NEG)
    m1 = jnp.max(masked, axis=1, keepdims=True)
    i1 = jnp.min(jnp.where(masked == m1, lane, E), axis=1, keepdims=True)
    masked2 = jnp.where(lane == i1, NEG, masked)
    m2 = jnp.max(masked2, axis=1, keepdims=True)
    i2 = jnp.min(jnp.where(masked2 == m2, lane, E), axis=1, keepdims=True)
    selmask = jnp.logical_or(lane == i1, lane == i2)
    wsel = jnp.where(selmask, scores, 0.0)
    wsum = jnp.sum(wsel, axis=1, keepdims=True) + 1e-20
    comb_ref[...] = wsel * (RSF / wsum)
    idx_ref[...] = jnp.concatenate([i1, i2], axis=1)


def _router(x, gate_w, bias):
    gwt = jnp.zeros((D, 128), F32).at[:, :E].set(gate_w.T)
    bias2 = bias.reshape(1, E)
    BT = 512
    return pl.pallas_call(
        _router_body,
        grid=(T // BT,),
        in_specs=[
            pl.BlockSpec((BT, D), lambda b: (b, 0)),
            pl.BlockSpec((D, 128), lambda b: (0, 0)),
            pl.BlockSpec((1, E), lambda b: (0, 0)),
        ],
        out_specs=[
            pl.BlockSpec((BT, E), lambda b: (b, 0)),
            pl.BlockSpec((BT, K), lambda b: (b, 0)),
        ],
        out_shape=[
            jax.ShapeDtypeStruct((T, E), F32),
            jax.ShapeDtypeStruct((T, K), jnp.int32),
        ],
    )(x, gwt, bias2)


def _shared_body(x_ref, wgu_ref, wd_ref, out_ref):
    gu = jax.lax.dot_general(x_ref[...].astype(BF), wgu_ref[...].astype(BF),
                             NT, preferred_element_type=F32)
    h = jax.nn.silu(gu[:, :ISH]) * gu[:, ISH:]
    out_ref[...] = jax.lax.dot_general(h.astype(BF), wd_ref[...].astype(BF),
                                       NT, preferred_element_type=F32)


def _shared(x, w_gu, w_d):
    BT = 512
    return pl.pallas_call(
        _shared_body,
        grid=(T // BT,),
        in_specs=[
            pl.BlockSpec((BT, D), lambda b: (b, 0)),
            pl.BlockSpec((2 * ISH, D), lambda b: (0, 0)),
            pl.BlockSpec((D, ISH), lambda b: (0, 0)),
        ],
        out_specs=pl.BlockSpec((BT, D), lambda b: (b, 0)),
        out_shape=jax.ShapeDtypeStruct((T, D), F32),
    )(x, w_gu, w_d)


def _routed_body(x_ref, wgu_ref, wd_ref, comb_ref, shared_ref, out_ref):
    e = pl.program_id(0)
    b = pl.program_id(1)
    BT = 512
    rows = pl.ds(b * BT, BT)
    xb = x_ref[rows, :]
    lane = jax.lax.broadcasted_iota(jnp.int32, (BT, E), 1)
    col = jnp.sum(jnp.where(lane == e, comb_ref[rows, :], 0.0), axis=1,
                  keepdims=True)
    gu = jax.lax.dot_general(xb.astype(BF), wgu_ref[0].astype(BF),
                             NT, preferred_element_type=F32)
    h = jax.nn.silu(gu[:, :I]) * gu[:, I:] * col
    y = jax.lax.dot_general(h.astype(BF), wd_ref[0].astype(BF),
                            NT, preferred_element_type=F32)

    @pl.when(e == 0)
    def _():
        out_ref[rows, :] = shared_ref[rows, :] + y

    @pl.when(e > 0)
    def _():
        out_ref[rows, :] = out_ref[rows, :] + y


def _routed(x, w_gu, w_d, comb, shared_out):
    NB = 4
    return pl.pallas_call(
        _routed_body,
        grid=(E, NB),
        in_specs=[
            pl.BlockSpec((T, D), lambda e, b: (0, 0)),
            pl.BlockSpec((1, 2 * I, D), lambda e, b: (e, 0, 0)),
            pl.BlockSpec((1, D, I), lambda e, b: (e, 0, 0)),
            pl.BlockSpec((T, E), lambda e, b: (0, 0)),
            pl.BlockSpec((T, D), lambda e, b: (0, 0)),
        ],
        out_specs=pl.BlockSpec((T, D), lambda e, b: (0, 0)),
        out_shape=jax.ShapeDtypeStruct((T, D), F32),
    )(x, w_gu, w_d, comb, shared_out)


def kernel(x, max_num_tokens_per_gpu, gate_w, e_score_correction_bias,
           w_shared_gate_up, w_shared_down, w_expert_gate_up, w_expert_down):
    comb, _ = _router(x, gate_w, e_score_correction_bias)
    shared_out = _shared(x, w_shared_gate_up, w_shared_down)
    return _routed(x, w_expert_gate_up, w_expert_down, comb, shared_out)
